# Initial kernel scaffold; baseline (speedup 1.0000x reference)
import jax
import jax.numpy as jnp
from jax.experimental import pallas as pl


def kernel(x, A, W):
    raise NotImplementedError("write your pallas kernel here")


# speedup vs baseline: 1.0348x; 1.0348x over previous
import jax
import jax.numpy as jnp
from jax.experimental import pallas as pl
from jax.experimental.pallas import tpu as pltpu


def _body(x_ref, a_ref, w_ref, out_ref, h_ref):
    @pl.when(pl.program_id(0) == 0)
    def _():
        h_ref[...] = jnp.dot(
            x_ref[...], w_ref[...], preferred_element_type=jnp.float32
        )

    out_ref[...] = jnp.dot(
        a_ref[...], h_ref[...], preferred_element_type=jnp.float32
    )


def kernel(x, A, W):
    N, d_in = x.shape
    d_out = W.shape[1]

    bm = 8
    for cand in range(8, min(N, 2048) + 1, 8):
        if N % cand == 0 and cand * N * 4 * 2 <= 64 * 1024 * 1024:
            bm = cand

    return pl.pallas_call(
        _body,
        grid=(N // bm,),
        in_specs=[
            pl.BlockSpec((N, d_in), lambda i: (0, 0)),
            pl.BlockSpec((bm, N), lambda i: (i, 0)),
            pl.BlockSpec((d_in, d_out), lambda i: (0, 0)),
        ],
        out_specs=pl.BlockSpec((bm, d_out), lambda i: (i, 0)),
        out_shape=jax.ShapeDtypeStruct((N, d_out), jnp.float32),
        scratch_shapes=[pltpu.VMEM((N, d_out), jnp.float32)],
    )(x, A, W)


# speedup vs baseline: 1.0351x; 1.0003x over previous
import jax
import jax.numpy as jnp
from jax.experimental import pallas as pl
from jax.experimental.pallas import tpu as pltpu


def _body(x_ref, a_ref, w_ref, out_ref, h_ref):
    @pl.when(pl.program_id(0) == 0)
    def _():
        h_ref[...] = jnp.dot(
            x_ref[...], w_ref[...], preferred_element_type=jnp.float32
        )

    out_ref[...] = jnp.dot(
        a_ref[...], h_ref[...], preferred_element_type=jnp.float32
    )


def kernel(x, A, W):
    N, d_in = x.shape
    d_out = W.shape[1]

    bm = 8
    for cand in range(8, min(N, 2048) + 1, 8):
        if N % cand == 0 and cand * N * 4 * 2 <= 26 * 1024 * 1024:
            bm = cand

    return pl.pallas_call(
        _body,
        grid=(N // bm,),
        in_specs=[
            pl.BlockSpec((N, d_in), lambda i: (0, 0)),
            pl.BlockSpec((bm, N), lambda i: (i, 0)),
            pl.BlockSpec((d_in, d_out), lambda i: (0, 0)),
        ],
        out_specs=pl.BlockSpec((bm, d_out), lambda i: (i, 0)),
        out_shape=jax.ShapeDtypeStruct((N, d_out), jnp.float32),
        scratch_shapes=[pltpu.VMEM((N, d_out), jnp.float32)],
    )(x, A, W)
